# baseline (speedup 1.0000x reference)
import jax
import jax.numpy as jnp
from jax.experimental import pallas as pl
from jax.experimental.pallas import tpu as pltpu

_W0 = 1.0 / 1223
_W1 = 1.0 / 2444
_W2 = 1.0 / 1687


def _ce_kernel(x_ref, t_ref, loss_ref):
    x = x_ref[...]
    t = t_ref[...]
    e = jnp.exp(x)
    lse = jnp.log(e[0:1, :] + e[1:2, :] + e[2:3, :])
    is0 = t == 0
    is1 = t == 1
    picked = jnp.where(is0, x[0:1, :], jnp.where(is1, x[1:2, :], x[2:3, :]))
    w = jnp.where(is0, _W0, jnp.where(is1, _W1, _W2)).astype(jnp.float32)
    num = jnp.sum(w * (lse - picked))
    den = jnp.sum(w)
    loss_ref[0, 0] = num / den


def kernel(index, output, target, pred_hist):
    del index, pred_hist
    x = output.T
    t = target.reshape(1, 16384)
    loss = pl.pallas_call(
        _ce_kernel,
        out_shape=jax.ShapeDtypeStruct((1, 1), jnp.float32),
        out_specs=pl.BlockSpec(memory_space=pltpu.SMEM),
    )(x, t)
    return loss[0, 0]
